# Initial kernel scaffold; baseline (speedup 1.0000x reference)
import jax
import jax.numpy as jnp
from jax.experimental import pallas as pl


def kernel(x_article, x_software, edge_index_references, edge_index_related, edge_index_mentioned_in, W_gcn_ref, b_gcn_ref, W_gcn_rel, b_gcn_rel, W_sage_l, b_sage_l, W_sage_r):
    raise NotImplementedError("write your pallas kernel here")


# speedup vs baseline: 10.2645x; 10.2645x over previous
import functools

import jax
import jax.numpy as jnp
from jax import lax
from jax.experimental import pallas as pl
from jax.experimental.pallas import tpu as pltpu
from jax.experimental.pallas import tpu_sc as plsc

N = 10000
D = 128
E = 320000
NC, NS = 2, 16
NW = NC * NS
CH = 128
CPT = 79
EPT = CH * CPT
EH = EPT * NS
E_PAD = EH * NC
N_ACC = 10240
SLAB = N_ACC // NS
ZR = 160


def _mesh():
    return plsc.VectorSubcoreMesh(core_axis_name="c", subcore_axis_name="s")


def _hist_body(d0, d1, d2, out, deg0, deg1, deg2, ones_v, zv, dst_v):
    cid = lax.axis_index("c")
    sid = lax.axis_index("s")
    vec1 = jnp.where(
        lax.broadcasted_iota(jnp.int32, (16,), 0) == 0,
        jnp.float32(1.0), jnp.float32(0.0))
    z16 = jnp.zeros((16,), jnp.float32)

    def init_ones(i, c):
        ones_v[i, :] = vec1
        return c
    lax.fori_loop(0, CH, init_ones, 0)

    def init_z(i, c):
        zv[i, :] = z16
        return c
    lax.fori_loop(0, SLAB, init_z, 0)

    row0 = sid * SLAB
    for dg in (deg0, deg1, deg2):
        pltpu.sync_copy(zv, dg.at[pl.ds(row0, SLAB), :])
    plsc.subcore_barrier()

    base = cid * EH + sid * EPT
    for dh, dg in ((d0, deg0), (d1, deg1), (d2, deg2)):
        def chunk(ci, c, dh=dh, dg=dg):
            off = pl.multiple_of(base + ci * CH, 8)
            pltpu.sync_copy(dh.at[pl.ds(off, CH)], dst_v)
            pltpu.sync_copy(ones_v, dg.at[dst_v], add=True)
            return c
        lax.fori_loop(0, CPT, chunk, 0)
    plsc.subcore_barrier()

    for r, dg in enumerate((deg0, deg1, deg2)):
        q = (cid * 3 + r) * N_ACC + row0
        pltpu.sync_copy(dg.at[pl.ds(row0, SLAB), :], out.at[pl.ds(q, SLAB), :])


def _hist_call(d0, d1, d2):
    f = pl.kernel(
        _hist_body,
        out_type=jax.ShapeDtypeStruct((NC * 3 * N_ACC, 16), jnp.float32),
        mesh=_mesh(),
        scratch_types=[
            pltpu.VMEM_SHARED((N_ACC, 16), jnp.float32),
            pltpu.VMEM_SHARED((N_ACC, 16), jnp.float32),
            pltpu.VMEM_SHARED((N_ACC, 16), jnp.float32),
            pltpu.VMEM((CH, 16), jnp.float32),
            pltpu.VMEM((SLAB, 16), jnp.float32),
            pltpu.VMEM((CH,), jnp.int32),
        ],
    )
    return f(d0, d1, d2)


def _scatter_body(t0, t1, t2, s0, d0, s1, d1, s2, d2, out,
                  acc, zacc, src_v, dst_v, rows_v, sem):
    cid = lax.axis_index("c")
    sid = lax.axis_index("s")
    z16 = jnp.zeros((16,), jnp.float32)

    def init_z(k, c):
        zacc[k // 8, pl.ds((k % 8) * 16, 16)] = z16
        return c
    lax.fori_loop(0, ZR * 8, init_z, 0)

    row0 = sid * SLAB
    base = cid * EH + sid * EPT
    for r, (tb, sh, dh) in enumerate(((t0, s0, d0), (t1, s1, d1), (t2, s2, d2))):
        for j in range(SLAB // ZR):
            pltpu.sync_copy(zacc, acc.at[pl.ds(row0 + j * ZR, ZR), :])
        plsc.subcore_barrier()

        def chunk(ci, c, tb=tb, sh=sh, dh=dh):
            off = pl.multiple_of(base + ci * CH, 8)
            pltpu.sync_copy(sh.at[pl.ds(off, CH)], src_v)
            pltpu.sync_copy(dh.at[pl.ds(off, CH)], dst_v)
            pltpu.async_copy(tb.at[src_v], rows_v, sem).wait()
            pltpu.sync_copy(rows_v, acc.at[dst_v], add=True)
            return c
        lax.fori_loop(0, CPT, chunk, 0)
        plsc.subcore_barrier()

        q = (r * NC + cid) * N_ACC + row0
        pltpu.sync_copy(acc.at[pl.ds(row0, SLAB), :], out.at[pl.ds(q, SLAB), :])


def _scatter_call(t0, t1, t2, s0, d0, s1, d1, s2, d2):
    f = pl.kernel(
        _scatter_body,
        out_type=jax.ShapeDtypeStruct((3 * NC * N_ACC, D), jnp.float32),
        mesh=_mesh(),
        scratch_types=[
            pltpu.VMEM_SHARED((N_ACC, D), jnp.float32),
            pltpu.VMEM((ZR, D), jnp.float32),
            pltpu.VMEM((CH,), jnp.int32),
            pltpu.VMEM((CH,), jnp.int32),
            pltpu.VMEM((CH, D), jnp.float32),
            pltpu.SemaphoreType.DMA,
        ],
    )
    return f(t0, t1, t2, s0, d0, s1, d1, s2, d2)


def _scale_body(xa, xs, dga, dgs, oa, os_):
    oa[...] = xa[...] * lax.rsqrt(dga[...])
    os_[...] = xs[...] * lax.rsqrt(dgs[...])


def _scale_call(x_article, x_software, deg_ref_col, deg_rel_col):
    nb = N // 1000
    row = pl.BlockSpec((1000, D), lambda i: (i, 0))
    col = pl.BlockSpec((1000, 1), lambda i: (i, 0))
    return pl.pallas_call(
        _scale_body,
        grid=(nb,),
        in_specs=[row, row, col, col],
        out_specs=[row, row],
        out_shape=[jax.ShapeDtypeStruct((N, D), jnp.float32)] * 2,
    )(x_article, x_software, deg_ref_col, deg_rel_col)


def _final_body(accs, xa, xs, dga, dgs, cnt, w_ref, w_rel, w_l, w_r,
                b_ref, b_rel, b_l, out_a, out_s):
    f32 = jnp.float32
    acc_ref = accs[0, 0] + accs[0, 1]
    acc_rel = accs[1, 0] + accs[1, 1]
    acc_men = accs[2, 0] + accs[2, 1]
    dinv_a = lax.rsqrt(dga[...])
    dinv_s = lax.rsqrt(dgs[...])
    xa_b = xa[...]
    xs_b = xs[...]
    gcn = jnp.dot(dinv_a * (acc_ref + dinv_a * xa_b), w_ref[...],
                  preferred_element_type=f32) + b_ref[...]
    mean = acc_men / jnp.maximum(cnt[...], 1.0)
    sage = (jnp.dot(mean, w_l[...], preferred_element_type=f32) + b_l[...]
            + jnp.dot(xa_b, w_r[...], preferred_element_type=f32))
    out_a[...] = jnp.maximum(0.5 * (gcn + sage), 0.0)
    gcn_s = jnp.dot(dinv_s * (acc_rel + dinv_s * xs_b), w_rel[...],
                    preferred_element_type=f32) + b_rel[...]
    out_s[...] = jnp.maximum(gcn_s, 0.0)


def _final_call(accs, x_article, x_software, deg_ref_col, deg_rel_col, cnt_col,
                w_ref, w_rel, w_l, w_r, b_ref, b_rel, b_l):
    nb = N // 1000
    row = pl.BlockSpec((1000, D), lambda i: (i, 0))
    col = pl.BlockSpec((1000, 1), lambda i: (i, 0))
    wsp = pl.BlockSpec((D, D), lambda i: (0, 0))
    bsp = pl.BlockSpec((1, D), lambda i: (0, 0))
    asp = pl.BlockSpec((3, NC, 1000, D), lambda i: (0, 0, i, 0))
    return pl.pallas_call(
        _final_body,
        grid=(nb,),
        in_specs=[asp, row, row, col, col, col, wsp, wsp, wsp, wsp,
                  bsp, bsp, bsp],
        out_specs=[row, row],
        out_shape=[jax.ShapeDtypeStruct((N, D), jnp.float32)] * 2,
    )(accs, x_article, x_software, deg_ref_col, deg_rel_col, cnt_col,
      w_ref, w_rel, w_l, w_r, b_ref, b_rel, b_l)


def kernel(x_article, x_software, edge_index_references, edge_index_related,
           edge_index_mentioned_in, W_gcn_ref, b_gcn_ref, W_gcn_rel, b_gcn_rel,
           W_sage_l, b_sage_l, W_sage_r):
    pad0 = jnp.zeros((E_PAD - E,), jnp.int32)
    padn = jnp.full((E_PAD - E,), N, jnp.int32)
    s_ref = jnp.concatenate([edge_index_references[0], pad0])
    d_ref = jnp.concatenate([edge_index_references[1], padn])
    s_rel = jnp.concatenate([edge_index_related[0], pad0])
    d_rel = jnp.concatenate([edge_index_related[1], padn])
    s_men = jnp.concatenate([edge_index_mentioned_in[0], pad0])
    d_men = jnp.concatenate([edge_index_mentioned_in[1], padn])

    deg_parts = _hist_call(d_ref, d_rel, d_men)
    dp = deg_parts.reshape(NC, 3, N_ACC, 16)[:, :, :N, 0]
    hist = dp[0] + dp[1]
    deg_ref_col = (hist[0] + 1.0).reshape(N, 1)
    deg_rel_col = (hist[1] + 1.0).reshape(N, 1)
    cnt_col = hist[2].reshape(N, 1)

    xs_ref, xs_rel = _scale_call(x_article, x_software, deg_ref_col, deg_rel_col)

    acc = _scatter_call(xs_ref, xs_rel, x_software,
                        s_ref, d_ref, s_rel, d_rel, s_men, d_men)
    accs = acc.reshape(3, NC, N_ACC, D)

    out_a, out_s = _final_call(
        accs, x_article, x_software, deg_ref_col, deg_rel_col, cnt_col,
        W_gcn_ref, W_gcn_rel, W_sage_l, W_sage_r,
        b_gcn_ref.reshape(1, D), b_gcn_rel.reshape(1, D), b_sage_l.reshape(1, D))
    return out_a, out_s
